# baseline (speedup 1.0000x reference)
import jax
import jax.numpy as jnp
from jax import lax
from jax.experimental import pallas as pl
from jax.experimental.pallas import tpu as pltpu
from jax.experimental.pallas import tpu_sc as plsc

N = 10000
E = 160000
D_IN = 256
D_EDGE = 16
H = 256
HH = 128
OUT = 256
L_LAYERS = 3
N_PAD = 10240
TILES = 16
ROWS_PER_TILE = N_PAD // TILES
CH = 80
TILE_E = E // TILES
N_CHUNKS = TILE_E // CH


def _ln(x, g, b, eps=1e-5):
    mu = jnp.mean(x, axis=-1, keepdims=True)
    var = jnp.mean((x - mu) ** 2, axis=-1, keepdims=True)
    return (x - mu) * lax.rsqrt(var + eps) * g + b


def _node_enc_body(x_ref, w_ref, b_ref, h_ref):
    h_ref[...] = jnp.dot(x_ref[...], w_ref[...],
                         preferred_element_type=jnp.float32) + b_ref[...]


def _node_enc(x, w, b):
    blk = 1000
    return pl.pallas_call(
        _node_enc_body,
        grid=(N // blk,),
        in_specs=[
            pl.BlockSpec((blk, D_IN), lambda i: (i, 0)),
            pl.BlockSpec((D_IN, H), lambda i: (0, 0)),
            pl.BlockSpec((1, H), lambda i: (0, 0)),
        ],
        out_specs=pl.BlockSpec((blk, H), lambda i: (i, 0)),
        out_shape=jax.ShapeDtypeStruct((N, H), jnp.float32),
    )(x, w, b)


def _tprep_body(h_ref, g_ref, b_ref, t_ref):
    t = jax.nn.relu(_ln(h_ref[...], g_ref[...], b_ref[...]))
    t_ref[0] = t[:, :HH]
    t_ref[1] = t[:, HH:]


def _tprep(h, g, b):
    blk = 1000
    return pl.pallas_call(
        _tprep_body,
        grid=(N // blk,),
        in_specs=[
            pl.BlockSpec((blk, H), lambda i: (i, 0)),
            pl.BlockSpec((1, H), lambda i: (0, 0)),
            pl.BlockSpec((1, H), lambda i: (0, 0)),
        ],
        out_specs=pl.BlockSpec((2, blk, HH), lambda i: (0, i, 0)),
        out_shape=jax.ShapeDtypeStruct((2, N, HH), jnp.float32),
    )(h, g, b)


def _edge_enc_body(a_ref, w_ref, b_ref, ea_ref):
    ea = jnp.dot(a_ref[...], w_ref[...],
                 preferred_element_type=jnp.float32) + b_ref[...]
    ea_ref[0] = ea[:, :HH]
    ea_ref[1] = ea[:, HH:]


def _edge_enc(a, w, b):
    blk = 2000
    return pl.pallas_call(
        _edge_enc_body,
        grid=(E // blk,),
        in_specs=[
            pl.BlockSpec((blk, D_EDGE), lambda i: (i, 0)),
            pl.BlockSpec((D_EDGE, H), lambda i: (0, 0)),
            pl.BlockSpec((1, H), lambda i: (0, 0)),
        ],
        out_specs=pl.BlockSpec((2, blk, HH), lambda i: (0, i, 0)),
        out_shape=jax.ShapeDtypeStruct((2, E, HH), jnp.float32),
    )(a, w, b)


def _mlp_body(aggr_ref, cnt_ref, t_ref, h_ref,
              w1_ref, b1_ref, g1_ref, lb1_ref, w2_ref, b2_ref,
              gn_ref, bn_ref, h_out_ref, t_out_ref):
    cnt = cnt_ref[0][:, 0:1] + cnt_ref[1][:, 0:1]
    deg = jnp.maximum(cnt, 1.0)
    u0 = (aggr_ref[0] + 1e-7 * cnt) / deg + t_ref[0]
    u1 = (aggr_ref[1] + 1e-7 * cnt) / deg + t_ref[1]
    u = jnp.concatenate([u0, u1], axis=1)
    m = jnp.dot(u, w1_ref[...], preferred_element_type=jnp.float32) + b1_ref[...]
    m = jax.nn.relu(_ln(m, g1_ref[...], lb1_ref[...]))
    m = jnp.dot(m, w2_ref[...], preferred_element_type=jnp.float32) + b2_ref[...]
    h_new = h_ref[...] + m
    h_out_ref[...] = h_new
    t = jax.nn.relu(_ln(h_new, gn_ref[...], bn_ref[...]))
    t_out_ref[0] = t[:, :HH]
    t_out_ref[1] = t[:, HH:]


def _mlp(aggr, cnt, t, h, w1, b1, g1, lb1, w2, b2, gn, bn):
    blk = 1000
    return pl.pallas_call(
        _mlp_body,
        grid=(N // blk,),
        in_specs=[
            pl.BlockSpec((2, blk, HH), lambda i: (0, i, 0)),
            pl.BlockSpec((2, blk, HH), lambda i: (0, i, 0)),
            pl.BlockSpec((2, blk, HH), lambda i: (0, i, 0)),
            pl.BlockSpec((blk, H), lambda i: (i, 0)),
            pl.BlockSpec((H, 2 * H), lambda i: (0, 0)),
            pl.BlockSpec((1, 2 * H), lambda i: (0, 0)),
            pl.BlockSpec((1, 2 * H), lambda i: (0, 0)),
            pl.BlockSpec((1, 2 * H), lambda i: (0, 0)),
            pl.BlockSpec((2 * H, H), lambda i: (0, 0)),
            pl.BlockSpec((1, H), lambda i: (0, 0)),
            pl.BlockSpec((1, H), lambda i: (0, 0)),
            pl.BlockSpec((1, H), lambda i: (0, 0)),
        ],
        out_specs=[
            pl.BlockSpec((blk, H), lambda i: (i, 0)),
            pl.BlockSpec((2, blk, HH), lambda i: (0, i, 0)),
        ],
        out_shape=[
            jax.ShapeDtypeStruct((N, H), jnp.float32),
            jax.ShapeDtypeStruct((2, N, HH), jnp.float32),
        ],
    )(aggr, cnt, t, h, w1, b1, g1, lb1, w2, b2, gn, bn)


def _head_body(t_ref, w_ref, b_ref, o_ref):
    u = jnp.concatenate([t_ref[0], t_ref[1]], axis=1)
    o_ref[...] = jnp.dot(u, w_ref[...],
                         preferred_element_type=jnp.float32) + b_ref[...]


def _head(t, w, b):
    blk = 1000
    return pl.pallas_call(
        _head_body,
        grid=(N // blk,),
        in_specs=[
            pl.BlockSpec((2, blk, HH), lambda i: (0, i, 0)),
            pl.BlockSpec((H, OUT), lambda i: (0, 0)),
            pl.BlockSpec((1, OUT), lambda i: (0, 0)),
        ],
        out_specs=pl.BlockSpec((blk, OUT), lambda i: (i, 0)),
        out_shape=jax.ShapeDtypeStruct((N, OUT), jnp.float32),
    )(t, w, b)


def _sc_aggr_kernel():
    mesh = plsc.VectorSubcoreMesh(core_axis_name="c", subcore_axis_name="s")
    out_type = [jax.ShapeDtypeStruct((2 * N_PAD, HH), jnp.float32)]
    scratch = [
        pltpu.VMEM((CH,), jnp.int32),
        pltpu.VMEM((CH,), jnp.int32),
        pltpu.VMEM((1, CH), jnp.int32),
        pltpu.VMEM((1, CH), jnp.int32),
        pltpu.VMEM((CH, HH), jnp.float32),
        pltpu.VMEM((CH, HH), jnp.float32),
        pltpu.VMEM((CH, HH), jnp.float32),
        pltpu.SemaphoreType.DMA,
        pltpu.SemaphoreType.DMA,
        pltpu.SemaphoreType.DMA,
        pltpu.SemaphoreType.DMA,
        pltpu.SemaphoreType.DMA,
        pltpu.VMEM_SHARED((N_PAD, HH), jnp.float32),
    ]

    def body(t_hbm, ea_hbm, src_hbm, dst_hbm, out_hbm,
             sidx0, sidx1, didx0, didx1, trow0, trow1, erow,
             isem0, isem1, gsem0, gsem1, easem, aggr_sp):
        c = lax.axis_index("c")
        s = lax.axis_index("s")
        sidxs = (sidx0, sidx1)
        didxs = (didx0, didx1)
        trows = (trow0, trow1)
        isems = (isem0, isem1)
        gsems = (gsem0, gsem1)

        zeros16 = jnp.zeros((16,), jnp.float32)
        for e in range(CH):
            for v in range(HH // 16):
                trow0[e, pl.ds(v * 16, 16)] = zeros16
        for k in range(ROWS_PER_TILE // CH):
            pltpu.sync_copy(trow0, aggr_sp.at[pl.ds(s * ROWS_PER_TILE + k * CH, CH)])
        plsc.subcore_barrier()

        ebase = s * TILE_E
        coff = c * N

        def fire_idx(j, b):
            pltpu.async_copy(src_hbm.at[pl.ds(ebase + j * CH, CH)],
                             sidxs[b], isems[b])
            pltpu.async_copy(dst_hbm.at[pl.ds(ebase + j * CH, CH)],
                             didxs[b].at[0], isems[b])

        def fire_idx_when(j, b):
            @pl.when(j < N_CHUNKS)
            def _():
                fire_idx(j, b)

        def fire_gather(j, b):
            pltpu.make_async_copy(src_hbm.at[pl.ds(0, CH)], sidxs[b],
                                  isems[b]).wait()
            pltpu.make_async_copy(dst_hbm.at[pl.ds(0, CH)], didxs[b].at[0],
                                  isems[b]).wait()
            for k in range(CH // 16):
                sl = pl.ds(k * 16, 16)
                sidxs[b][sl] = sidxs[b][sl] + coff
            pltpu.async_copy(t_hbm.at[sidxs[b]], trows[b], gsems[b])

        def fire_gather_when(j, b):
            @pl.when(j < N_CHUNKS)
            def _():
                fire_gather(j, b)

        def consume(j, b):
            pltpu.async_copy(ea_hbm.at[pl.ds(c * E + ebase + j * CH, CH)],
                             erow, easem)
            pltpu.make_async_copy(t_hbm.at[sidxs[b]], trows[b], gsems[b]).wait()
            pltpu.make_async_copy(ea_hbm.at[pl.ds(0, CH)], erow, easem).wait()
            tr = trows[b]

            def edge(e, ecarry):
                for v in range(HH // 16):
                    sl = pl.ds(v * 16, 16)
                    tr[e, sl] = jnp.maximum(tr[e, sl] + erow[e, sl], 0.0)
                return ecarry
            lax.fori_loop(0, CH, edge, 0, unroll=4)
            pltpu.sync_copy(tr, aggr_sp.at[didxs[b].at[0]], add=True)

        def consume_when(j, b):
            @pl.when(j < N_CHUNKS)
            def _():
                consume(j, b)

        fire_idx(0, 0)
        fire_idx(1, 1)
        fire_gather(0, 0)

        def pair(i, carry):
            j0 = i * 2
            fire_gather_when(j0 + 1, 1)
            consume(j0, 0)
            fire_idx_when(j0 + 2, 0)
            fire_gather_when(j0 + 2, 0)
            consume_when(j0 + 1, 1)
            fire_idx_when(j0 + 3, 1)
            return carry
        lax.fori_loop(0, (N_CHUNKS + 1) // 2, pair, 0)

        plsc.subcore_barrier()
        row0 = s * ROWS_PER_TILE
        pltpu.sync_copy(aggr_sp.at[pl.ds(row0, ROWS_PER_TILE)],
                        out_hbm.at[pl.ds(c * N_PAD + row0, ROWS_PER_TILE)])

    return pl.kernel(body, out_type=out_type, mesh=mesh, scratch_types=scratch)


def _sc_deg_kernel():
    EC = E // 2
    TEC = EC // TILES
    DCH = 40
    NDCH = TEC // DCH
    mesh = plsc.VectorSubcoreMesh(core_axis_name="c", subcore_axis_name="s")
    out_type = [jax.ShapeDtypeStruct((2 * N_PAD, HH), jnp.float32)]
    scratch = [
        pltpu.VMEM((DCH,), jnp.int32),
        pltpu.VMEM((DCH, HH), jnp.float32),
        pltpu.VMEM_SHARED((N_PAD, HH), jnp.float32),
    ]

    def body(dst_hbm, cnt_hbm, didx, ones, cnt_sp):
        c = lax.axis_index("c")
        s = lax.axis_index("s")
        zeros16 = jnp.zeros((16,), jnp.float32)
        for e in range(DCH):
            for v in range(HH // 16):
                ones[e, pl.ds(v * 16, 16)] = zeros16
        for k in range(ROWS_PER_TILE // DCH):
            pltpu.sync_copy(ones, cnt_sp.at[pl.ds(s * ROWS_PER_TILE + k * DCH, DCH)])
        one16 = jnp.full((16,), 1.0, jnp.float32)
        for e in range(DCH):
            for v in range(HH // 16):
                ones[e, pl.ds(v * 16, 16)] = one16
        plsc.subcore_barrier()

        def chunk(j, carry):
            base = c * EC + s * TEC + j * DCH
            pltpu.sync_copy(dst_hbm.at[pl.ds(base, DCH)], didx)
            pltpu.sync_copy(ones, cnt_sp.at[didx], add=True)
            return carry
        lax.fori_loop(0, NDCH, chunk, 0)
        plsc.subcore_barrier()
        row0 = s * ROWS_PER_TILE
        pltpu.sync_copy(cnt_sp.at[pl.ds(row0, ROWS_PER_TILE)],
                        cnt_hbm.at[pl.ds(c * N_PAD + row0, ROWS_PER_TILE)])

    return pl.kernel(body, out_type=out_type, mesh=mesh, scratch_types=scratch)


_SC_CACHE = {}


def _sc_aggr(*args):
    if "aggr" not in _SC_CACHE:
        _SC_CACHE["aggr"] = _sc_aggr_kernel()
    return _SC_CACHE["aggr"](*args)


def _sc_deg(*args):
    if "deg" not in _SC_CACHE:
        _SC_CACHE["deg"] = _sc_deg_kernel()
    return _SC_CACHE["deg"](*args)


def kernel(x, edge_index, edge_attr, W_node, b_node, W_edge, b_edge,
           ln_g, ln_b, W1, b1, lng1, lnb1, W2, b2,
           gamma_out, beta_out, W_out, b_out):
    src = edge_index[0]
    dst = edge_index[1]
    r1 = lambda v: v.reshape(1, -1)

    h = _node_enc(x, W_node, r1(b_node))
    ea = _edge_enc(edge_attr, W_edge, r1(b_edge)).reshape(2 * E, HH)
    t = _tprep(h, r1(ln_g[0]), r1(ln_b[0]))

    (cnt,) = _sc_deg(dst)
    cnt = cnt.reshape(2, N_PAD, HH)
    for i in range(L_LAYERS):
        t_flat = t.reshape(2 * N, HH)
        (aggr,) = _sc_aggr(t_flat, ea, src, dst)
        aggr = aggr.reshape(2, N_PAD, HH)
        if i < L_LAYERS - 1:
            gn, bn = r1(ln_g[i + 1]), r1(ln_b[i + 1])
        else:
            gn, bn = r1(gamma_out), r1(beta_out)
        h, t = _mlp(aggr, cnt, t, h,
                    W1[i], r1(b1[i]), r1(lng1[i]), r1(lnb1[i]),
                    W2[i], r1(b2[i]), gn, bn)

    return _head(t, W_out, r1(b_out))


# speedup vs baseline: 1.8086x; 1.8086x over previous
import jax
import jax.numpy as jnp
from jax import lax
from jax.experimental import pallas as pl
from jax.experimental.pallas import tpu as pltpu
from jax.experimental.pallas import tpu_sc as plsc

N = 10000
E = 160000
D_IN = 256
D_EDGE = 16
H = 256
HH = 128
OUT = 256
L_LAYERS = 3
N_PAD = 10240
TILES = 16
ROWS_PER_TILE = N_PAD // TILES
CH = 80
TILE_E = E // TILES
N_CHUNKS = TILE_E // CH


def _ln(x, g, b, eps=1e-5):
    mu = jnp.mean(x, axis=-1, keepdims=True)
    var = jnp.mean((x - mu) ** 2, axis=-1, keepdims=True)
    return (x - mu) * lax.rsqrt(var + eps) * g + b


def _node_enc_body(x_ref, w_ref, b_ref, h_ref):
    h_ref[...] = jnp.dot(x_ref[...], w_ref[...],
                         preferred_element_type=jnp.float32) + b_ref[...]


def _node_enc(x, w, b):
    blk = 1000
    return pl.pallas_call(
        _node_enc_body,
        grid=(N // blk,),
        in_specs=[
            pl.BlockSpec((blk, D_IN), lambda i: (i, 0)),
            pl.BlockSpec((D_IN, H), lambda i: (0, 0)),
            pl.BlockSpec((1, H), lambda i: (0, 0)),
        ],
        out_specs=pl.BlockSpec((blk, H), lambda i: (i, 0)),
        out_shape=jax.ShapeDtypeStruct((N, H), jnp.float32),
    )(x, w, b)


def _tprep_body(h_ref, g_ref, b_ref, t_ref):
    t = jax.nn.relu(_ln(h_ref[...], g_ref[...], b_ref[...]))
    t_ref[0] = t[:, :HH]
    t_ref[1] = t[:, HH:]


def _tprep(h, g, b):
    blk = 1000
    return pl.pallas_call(
        _tprep_body,
        grid=(N // blk,),
        in_specs=[
            pl.BlockSpec((blk, H), lambda i: (i, 0)),
            pl.BlockSpec((1, H), lambda i: (0, 0)),
            pl.BlockSpec((1, H), lambda i: (0, 0)),
        ],
        out_specs=pl.BlockSpec((2, blk, HH), lambda i: (0, i, 0)),
        out_shape=jax.ShapeDtypeStruct((2, N, HH), jnp.float32),
    )(h, g, b)


def _edge_enc_body(a_ref, w_ref, b_ref, ea_ref):
    ea = jnp.dot(a_ref[...], w_ref[...],
                 preferred_element_type=jnp.float32) + b_ref[...]
    ea_ref[0] = ea[:, :HH]
    ea_ref[1] = ea[:, HH:]


def _edge_enc(a, w, b):
    blk = 2000
    return pl.pallas_call(
        _edge_enc_body,
        grid=(E // blk,),
        in_specs=[
            pl.BlockSpec((blk, D_EDGE), lambda i: (i, 0)),
            pl.BlockSpec((D_EDGE, H), lambda i: (0, 0)),
            pl.BlockSpec((1, H), lambda i: (0, 0)),
        ],
        out_specs=pl.BlockSpec((2, blk, HH), lambda i: (0, i, 0)),
        out_shape=jax.ShapeDtypeStruct((2, E, HH), jnp.float32),
    )(a, w, b)


def _mlp_body(aggr_ref, cnt_ref, t_ref, h_ref,
              w1_ref, b1_ref, g1_ref, lb1_ref, w2_ref, b2_ref,
              gn_ref, bn_ref, h_out_ref, t_out_ref):
    cnt = cnt_ref[0][:, 0:1] + cnt_ref[1][:, 0:1]
    deg = jnp.maximum(cnt, 1.0)
    u0 = (aggr_ref[0] + 1e-7 * cnt) / deg + t_ref[0]
    u1 = (aggr_ref[1] + 1e-7 * cnt) / deg + t_ref[1]
    u = jnp.concatenate([u0, u1], axis=1)
    m = jnp.dot(u, w1_ref[...], preferred_element_type=jnp.float32) + b1_ref[...]
    m = jax.nn.relu(_ln(m, g1_ref[...], lb1_ref[...]))
    m = jnp.dot(m, w2_ref[...], preferred_element_type=jnp.float32) + b2_ref[...]
    h_new = h_ref[...] + m
    h_out_ref[...] = h_new
    t = jax.nn.relu(_ln(h_new, gn_ref[...], bn_ref[...]))
    t_out_ref[0] = t[:, :HH]
    t_out_ref[1] = t[:, HH:]


def _mlp(aggr, cnt, t, h, w1, b1, g1, lb1, w2, b2, gn, bn):
    blk = 1000
    return pl.pallas_call(
        _mlp_body,
        grid=(N // blk,),
        in_specs=[
            pl.BlockSpec((2, blk, HH), lambda i: (0, i, 0)),
            pl.BlockSpec((2, blk, HH), lambda i: (0, i, 0)),
            pl.BlockSpec((2, blk, HH), lambda i: (0, i, 0)),
            pl.BlockSpec((blk, H), lambda i: (i, 0)),
            pl.BlockSpec((H, 2 * H), lambda i: (0, 0)),
            pl.BlockSpec((1, 2 * H), lambda i: (0, 0)),
            pl.BlockSpec((1, 2 * H), lambda i: (0, 0)),
            pl.BlockSpec((1, 2 * H), lambda i: (0, 0)),
            pl.BlockSpec((2 * H, H), lambda i: (0, 0)),
            pl.BlockSpec((1, H), lambda i: (0, 0)),
            pl.BlockSpec((1, H), lambda i: (0, 0)),
            pl.BlockSpec((1, H), lambda i: (0, 0)),
        ],
        out_specs=[
            pl.BlockSpec((blk, H), lambda i: (i, 0)),
            pl.BlockSpec((2, blk, HH), lambda i: (0, i, 0)),
        ],
        out_shape=[
            jax.ShapeDtypeStruct((N, H), jnp.float32),
            jax.ShapeDtypeStruct((2, N, HH), jnp.float32),
        ],
    )(aggr, cnt, t, h, w1, b1, g1, lb1, w2, b2, gn, bn)


def _head_body(t_ref, w_ref, b_ref, o_ref):
    u = jnp.concatenate([t_ref[0], t_ref[1]], axis=1)
    o_ref[...] = jnp.dot(u, w_ref[...],
                         preferred_element_type=jnp.float32) + b_ref[...]


def _head(t, w, b):
    blk = 1000
    return pl.pallas_call(
        _head_body,
        grid=(N // blk,),
        in_specs=[
            pl.BlockSpec((2, blk, HH), lambda i: (0, i, 0)),
            pl.BlockSpec((H, OUT), lambda i: (0, 0)),
            pl.BlockSpec((1, OUT), lambda i: (0, 0)),
        ],
        out_specs=pl.BlockSpec((blk, OUT), lambda i: (i, 0)),
        out_shape=jax.ShapeDtypeStruct((N, OUT), jnp.float32),
    )(t, w, b)


def _sc_aggr_kernel():
    mesh = plsc.VectorSubcoreMesh(core_axis_name="c", subcore_axis_name="s")
    out_type = [jax.ShapeDtypeStruct((2 * N_PAD, HH), jnp.float32)]
    scratch = [
        pltpu.VMEM((CH,), jnp.int32),
        pltpu.VMEM((CH,), jnp.int32),
        pltpu.VMEM((CH,), jnp.int32),
        pltpu.VMEM((CH,), jnp.int32),
        pltpu.VMEM((1, CH), jnp.int32),
        pltpu.VMEM((1, CH), jnp.int32),
        pltpu.VMEM((1, CH), jnp.int32),
        pltpu.VMEM((1, CH), jnp.int32),
        pltpu.VMEM((CH, HH), jnp.float32),
        pltpu.VMEM((CH, HH), jnp.float32),
        pltpu.VMEM((CH, HH), jnp.float32),
        pltpu.SemaphoreType.DMA,
        pltpu.SemaphoreType.DMA,
        pltpu.SemaphoreType.DMA,
        pltpu.SemaphoreType.DMA,
        pltpu.SemaphoreType.DMA,
        pltpu.SemaphoreType.DMA,
        pltpu.SemaphoreType.DMA,
        pltpu.VMEM_SHARED((N_PAD, HH), jnp.float32),
    ]

    def body(t_hbm, ea_hbm, src_hbm, dst_hbm, out_hbm,
             sidx0, sidx1, sidx2, sidx3, didx0, didx1, didx2, didx3,
             trow0, trow1, erow,
             isem0, isem1, isem2, isem3, gsem0, gsem1, easem, aggr_sp):
        c = lax.axis_index("c")
        s = lax.axis_index("s")
        sidxs = (sidx0, sidx1, sidx2, sidx3)
        didxs = (didx0, didx1, didx2, didx3)
        trows = (trow0, trow1)
        isems = (isem0, isem1, isem2, isem3)
        gsems = (gsem0, gsem1)

        zeros16 = jnp.zeros((16,), jnp.float32)
        for e in range(CH):
            for v in range(HH // 16):
                trow0[e, pl.ds(v * 16, 16)] = zeros16
        for k in range(ROWS_PER_TILE // CH):
            pltpu.sync_copy(trow0, aggr_sp.at[pl.ds(s * ROWS_PER_TILE + k * CH, CH)])
        plsc.subcore_barrier()

        ebase = s * TILE_E
        coff = c * N

        def fire_idx(j, q):
            pltpu.async_copy(src_hbm.at[pl.ds(ebase + j * CH, CH)],
                             sidxs[q], isems[q])
            pltpu.async_copy(dst_hbm.at[pl.ds(ebase + j * CH, CH)],
                             didxs[q].at[0], isems[q])

        def fire_gather(j, b, q):
            pltpu.make_async_copy(src_hbm.at[pl.ds(0, CH)], sidxs[q],
                                  isems[q]).wait()
            pltpu.make_async_copy(dst_hbm.at[pl.ds(0, CH)], didxs[q].at[0],
                                  isems[q]).wait()
            for k in range(CH // 16):
                sl = pl.ds(k * 16, 16)
                sidxs[q][sl] = sidxs[q][sl] + coff
            pltpu.async_copy(t_hbm.at[sidxs[q]], trows[b], gsems[b])

        def consume(j, b, q):
            pltpu.async_copy(ea_hbm.at[pl.ds(c * E + ebase + j * CH, CH)],
                             erow, easem)
            pltpu.make_async_copy(t_hbm.at[sidxs[q]], trows[b], gsems[b]).wait()
            pltpu.make_async_copy(ea_hbm.at[pl.ds(0, CH)], erow, easem).wait()
            tr = trows[b]

            def edge(e, ecarry):
                for v in range(HH // 16):
                    sl = pl.ds(v * 16, 16)
                    tr[e, sl] = jnp.maximum(tr[e, sl] + erow[e, sl], 0.0)
                return ecarry
            lax.fori_loop(0, CH, edge, 0)
            pltpu.sync_copy(tr, aggr_sp.at[didxs[q].at[0]], add=True)

        fire_idx(0, 0)
        fire_idx(1, 1)
        fire_idx(2, 2)
        fire_idx(3, 3)
        fire_gather(0, 0, 0)

        def quad(i, carry):
            j0 = i * 4
            fire_gather(j0 + 1, 1, 1)
            consume(j0, 0, 0)
            fire_idx(j0 + 4, 0)
            fire_gather(j0 + 2, 0, 2)
            consume(j0 + 1, 1, 1)
            fire_idx(j0 + 5, 1)
            fire_gather(j0 + 3, 1, 3)
            consume(j0 + 2, 0, 2)
            fire_idx(j0 + 6, 2)
            fire_gather(j0 + 4, 0, 0)
            consume(j0 + 3, 1, 3)
            fire_idx(j0 + 7, 3)
            return carry
        lax.fori_loop(0, (N_CHUNKS - 5) // 4, quad, 0)

        j0 = N_CHUNKS - 5
        fire_gather(j0 + 1, 1, 1)
        consume(j0, 0, 0)
        fire_idx(j0 + 4, 0)
        fire_gather(j0 + 2, 0, 2)
        consume(j0 + 1, 1, 1)
        fire_gather(j0 + 3, 1, 3)
        consume(j0 + 2, 0, 2)
        fire_gather(j0 + 4, 0, 0)
        consume(j0 + 3, 1, 3)
        consume(j0 + 4, 0, 0)

        plsc.subcore_barrier()
        row0 = s * ROWS_PER_TILE
        pltpu.sync_copy(aggr_sp.at[pl.ds(row0, ROWS_PER_TILE)],
                        out_hbm.at[pl.ds(c * N_PAD + row0, ROWS_PER_TILE)])

    return pl.kernel(body, out_type=out_type, mesh=mesh, scratch_types=scratch)


def _sc_deg_kernel():
    EC = E // 2
    TEC = EC // TILES
    DCH = 40
    NDCH = TEC // DCH
    mesh = plsc.VectorSubcoreMesh(core_axis_name="c", subcore_axis_name="s")
    out_type = [jax.ShapeDtypeStruct((2 * N_PAD, HH), jnp.float32)]
    scratch = [
        pltpu.VMEM((DCH,), jnp.int32),
        pltpu.VMEM((DCH, HH), jnp.float32),
        pltpu.VMEM_SHARED((N_PAD, HH), jnp.float32),
    ]

    def body(dst_hbm, cnt_hbm, didx, ones, cnt_sp):
        c = lax.axis_index("c")
        s = lax.axis_index("s")
        zeros16 = jnp.zeros((16,), jnp.float32)
        for e in range(DCH):
            for v in range(HH // 16):
                ones[e, pl.ds(v * 16, 16)] = zeros16
        for k in range(ROWS_PER_TILE // DCH):
            pltpu.sync_copy(ones, cnt_sp.at[pl.ds(s * ROWS_PER_TILE + k * DCH, DCH)])
        one16 = jnp.full((16,), 1.0, jnp.float32)
        for e in range(DCH):
            for v in range(HH // 16):
                ones[e, pl.ds(v * 16, 16)] = one16
        plsc.subcore_barrier()

        def chunk(j, carry):
            base = c * EC + s * TEC + j * DCH
            pltpu.sync_copy(dst_hbm.at[pl.ds(base, DCH)], didx)
            pltpu.sync_copy(ones, cnt_sp.at[didx], add=True)
            return carry
        lax.fori_loop(0, NDCH, chunk, 0)
        plsc.subcore_barrier()
        row0 = s * ROWS_PER_TILE
        pltpu.sync_copy(cnt_sp.at[pl.ds(row0, ROWS_PER_TILE)],
                        cnt_hbm.at[pl.ds(c * N_PAD + row0, ROWS_PER_TILE)])

    return pl.kernel(body, out_type=out_type, mesh=mesh, scratch_types=scratch)


_SC_CACHE = {}


def _sc_aggr(*args):
    if "aggr" not in _SC_CACHE:
        _SC_CACHE["aggr"] = _sc_aggr_kernel()
    return _SC_CACHE["aggr"](*args)


def _sc_deg(*args):
    if "deg" not in _SC_CACHE:
        _SC_CACHE["deg"] = _sc_deg_kernel()
    return _SC_CACHE["deg"](*args)


def kernel(x, edge_index, edge_attr, W_node, b_node, W_edge, b_edge,
           ln_g, ln_b, W1, b1, lng1, lnb1, W2, b2,
           gamma_out, beta_out, W_out, b_out):
    src = edge_index[0]
    dst = edge_index[1]
    r1 = lambda v: v.reshape(1, -1)

    h = _node_enc(x, W_node, r1(b_node))
    ea = _edge_enc(edge_attr, W_edge, r1(b_edge)).reshape(2 * E, HH)
    t = _tprep(h, r1(ln_g[0]), r1(ln_b[0]))

    (cnt,) = _sc_deg(dst)
    cnt = cnt.reshape(2, N_PAD, HH)
    for i in range(L_LAYERS):
        t_flat = t.reshape(2 * N, HH)
        (aggr,) = _sc_aggr(t_flat, ea, src, dst)
        aggr = aggr.reshape(2, N_PAD, HH)
        if i < L_LAYERS - 1:
            gn, bn = r1(ln_g[i + 1]), r1(ln_b[i + 1])
        else:
            gn, bn = r1(gamma_out), r1(beta_out)
        h, t = _mlp(aggr, cnt, t, h,
                    W1[i], r1(b1[i]), r1(lng1[i]), r1(lnb1[i]),
                    W2[i], r1(b2[i]), gn, bn)

    return _head(t, W_out, r1(b_out))
